# Initial kernel scaffold; baseline (speedup 1.0000x reference)
import jax
import jax.numpy as jnp
from jax.experimental import pallas as pl


def kernel(x_target, W, b, document_embeddings):
    raise NotImplementedError("write your pallas kernel here")


# speedup vs baseline: 1.1412x; 1.1412x over previous
import functools

import jax
import jax.numpy as jnp
from jax.experimental import pallas as pl
from jax.experimental.pallas import tpu as pltpu

TOPK = 8


def _linear_kernel(xm_ref, w_ref, b_ref, o_ref):
    o_ref[...] = jax.lax.dot_general(
        xm_ref[...], w_ref[...], (((1,), (1,)), ((), ())),
        preferred_element_type=jnp.float32) + b_ref[...]


def _main_kernel(qn_ref, d_ref, vals_ref, idxs_ref, cv_ref, ci_ref, *,
                 k_total, k_blk):
    kb = pl.program_id(1)
    nkb = pl.num_programs(1)

    q = qn_ref[...]
    dn = d_ref[...]
    s = jax.lax.dot_general(q, dn, (((1,), (1,)), ((), ())),
                            preferred_element_type=jnp.float32)
    ii = jax.lax.broadcasted_iota(jnp.int32, s.shape, 1) + kb * k_blk
    s = jnp.where(ii < k_total, s, -jnp.inf)

    bvals, bidx = [], []
    big = jnp.int32(2 ** 30)
    for t in range(TOPK):
        m = jnp.max(s, axis=1)
        am = jnp.min(jnp.where(s == m[:, None], ii, big), axis=1)
        bvals.append(m)
        bidx.append(am)
        if t < TOPK - 1:
            s = jnp.where(ii == am[:, None], -jnp.inf, s)
    bv = jnp.stack(bvals, axis=1)
    bi = jnp.stack(bidx, axis=1)

    @pl.when(kb == 0)
    def _init_carry():
        cv_ref[...] = jnp.full(cv_ref.shape, -jnp.inf, jnp.float32)
        ci_ref[...] = jnp.zeros(ci_ref.shape, jnp.int32)

    allv = jnp.concatenate([cv_ref[...], bv], axis=1)
    alli = jnp.concatenate([ci_ref[...], bi], axis=1)
    jj = jax.lax.broadcasted_iota(jnp.int32, allv.shape, 1)
    nv, ni = [], []
    for t in range(TOPK):
        m = jnp.max(allv, axis=1)
        p = jnp.min(jnp.where(allv == m[:, None], jj, big), axis=1)
        sel = jj == p[:, None]
        nv.append(m)
        ni.append(jnp.sum(jnp.where(sel, alli, 0), axis=1))
        if t < TOPK - 1:
            allv = jnp.where(sel, -jnp.inf, allv)
    cv_ref[...] = jnp.stack(nv, axis=1)
    ci_ref[...] = jnp.stack(ni, axis=1)

    @pl.when(kb == nkb - 1)
    def _emit():
        vals_ref[...] = cv_ref[...]
        idxs_ref[...] = ci_ref[...]


@jax.jit
def kernel(x_target, W, b, document_embeddings):
    Q, S, D = x_target.shape
    K = document_embeddings.shape[0]

    Q_BLK = 256
    K_BLK = 2048
    kpad = ((K + K_BLK - 1) // K_BLK) * K_BLK

    xm = jnp.mean(x_target, axis=1)

    q = pl.pallas_call(
        _linear_kernel,
        grid=(Q // 128,),
        in_specs=[
            pl.BlockSpec((128, D), lambda qb: (qb, 0)),
            pl.BlockSpec((D, D), lambda qb: (0, 0)),
            pl.BlockSpec((1, D), lambda qb: (0, 0)),
        ],
        out_specs=pl.BlockSpec((128, D), lambda qb: (qb, 0)),
        out_shape=jax.ShapeDtypeStruct((Q, D), jnp.float32),
    )(xm, W, b.reshape(1, D))

    qn = q / (jnp.linalg.norm(q, axis=-1, keepdims=True) + 1e-12)

    de = document_embeddings / (
        jnp.linalg.norm(document_embeddings, axis=-1, keepdims=True) + 1e-12)
    de = de / (jnp.linalg.norm(de, axis=-1, keepdims=True) + 1e-12)
    de = jnp.pad(de, ((0, kpad - K), (0, 0)))

    vals, idxs = pl.pallas_call(
        functools.partial(_main_kernel, k_total=K, k_blk=K_BLK),
        grid=(Q // Q_BLK, kpad // K_BLK),
        in_specs=[
            pl.BlockSpec((Q_BLK, D), lambda qb, kb: (qb, 0)),
            pl.BlockSpec((K_BLK, D), lambda qb, kb: (kb, 0)),
        ],
        out_specs=[
            pl.BlockSpec((Q_BLK, TOPK), lambda qb, kb: (qb, 0)),
            pl.BlockSpec((Q_BLK, TOPK), lambda qb, kb: (qb, 0)),
        ],
        out_shape=[
            jax.ShapeDtypeStruct((Q, TOPK), jnp.float32),
            jax.ShapeDtypeStruct((Q, TOPK), jnp.int32),
        ],
        scratch_shapes=[
            pltpu.VMEM((Q_BLK, TOPK), jnp.float32),
            pltpu.VMEM((Q_BLK, TOPK), jnp.int32),
        ],
    )(qn, de)

    return vals, idxs


# speedup vs baseline: 1.4417x; 1.2633x over previous
import functools

import jax
import jax.numpy as jnp
from jax.experimental import pallas as pl
from jax.experimental.pallas import tpu as pltpu

TOPK = 8


def _linear_kernel(xm_ref, w_ref, b_ref, o_ref):
    o_ref[...] = jax.lax.dot_general(
        xm_ref[...], w_ref[...], (((1,), (1,)), ((), ())),
        preferred_element_type=jnp.float32) + b_ref[...]


def _topk_rounds(s, ii, n, big):
    vals, idxs = [], []
    for t in range(n):
        m = jnp.max(s, axis=1)
        am = jnp.min(jnp.where(s == m[:, None], ii, big), axis=1)
        vals.append(m)
        idxs.append(am)
        if t < n - 1:
            s = jnp.where(ii == am[:, None], -jnp.inf, s)
    return jnp.stack(vals, axis=1), jnp.stack(idxs, axis=1)


def _main_kernel(qn_ref, d_ref, vals_ref, idxs_ref, cv_ref, ci_ref, *,
                 k_total, k_blk):
    kb = pl.program_id(1)
    nkb = pl.num_programs(1)
    big = jnp.int32(2 ** 30)

    q = qn_ref[...]
    dn = d_ref[...]
    s = jax.lax.dot_general(q, dn, (((1,), (1,)), ((), ())),
                            preferred_element_type=jnp.float32)
    ii = jax.lax.broadcasted_iota(jnp.int32, s.shape, 1) + kb * k_blk
    s = jnp.where(ii < k_total, s, -jnp.inf)

    bv, bi = _topk_rounds(s, ii, TOPK, big)
    qb_rows = s.shape[0]
    fill_v = jnp.full((qb_rows, 128 - TOPK), -jnp.inf, jnp.float32)
    fill_i = jnp.zeros((qb_rows, 128 - TOPK), jnp.int32)
    off = pl.multiple_of(kb * 128, 128)
    cv_ref[:, pl.ds(off, 128)] = jnp.concatenate([bv, fill_v], axis=1)
    ci_ref[:, pl.ds(off, 128)] = jnp.concatenate([bi, fill_i], axis=1)

    @pl.when(kb == nkb - 1)
    def _emit():
        allv = cv_ref[...]
        alli = ci_ref[...]
        jj = jax.lax.broadcasted_iota(jnp.int32, allv.shape, 1)
        nv, ni = [], []
        for t in range(TOPK):
            m = jnp.max(allv, axis=1)
            p = jnp.min(jnp.where(allv == m[:, None], jj, big), axis=1)
            sel = jj == p[:, None]
            nv.append(m)
            ni.append(jnp.sum(jnp.where(sel, alli, 0), axis=1))
            if t < TOPK - 1:
                allv = jnp.where(sel, -jnp.inf, allv)
        vals_ref[...] = jnp.stack(nv, axis=1)
        idxs_ref[...] = jnp.stack(ni, axis=1)


@jax.jit
def kernel(x_target, W, b, document_embeddings):
    Q, S, D = x_target.shape
    K = document_embeddings.shape[0]

    Q_BLK = 256
    K_BLK = 2048
    kpad = ((K + K_BLK - 1) // K_BLK) * K_BLK
    nkb = kpad // K_BLK

    xm = jnp.mean(x_target, axis=1)

    q = pl.pallas_call(
        _linear_kernel,
        grid=(Q // 128,),
        in_specs=[
            pl.BlockSpec((128, D), lambda qb: (qb, 0)),
            pl.BlockSpec((D, D), lambda qb: (0, 0)),
            pl.BlockSpec((1, D), lambda qb: (0, 0)),
        ],
        out_specs=pl.BlockSpec((128, D), lambda qb: (qb, 0)),
        out_shape=jax.ShapeDtypeStruct((Q, D), jnp.float32),
    )(xm, W, b.reshape(1, D))

    qn = q / (jnp.linalg.norm(q, axis=-1, keepdims=True) + 1e-12)

    de = document_embeddings / (
        jnp.linalg.norm(document_embeddings, axis=-1, keepdims=True) + 1e-12)
    de = de / (jnp.linalg.norm(de, axis=-1, keepdims=True) + 1e-12)
    de = jnp.pad(de, ((0, kpad - K), (0, 0)))

    vals, idxs = pl.pallas_call(
        functools.partial(_main_kernel, k_total=K, k_blk=K_BLK),
        grid=(Q // Q_BLK, nkb),
        in_specs=[
            pl.BlockSpec((Q_BLK, D), lambda qb, kb: (qb, 0)),
            pl.BlockSpec((K_BLK, D), lambda qb, kb: (kb, 0)),
        ],
        out_specs=[
            pl.BlockSpec((Q_BLK, TOPK), lambda qb, kb: (qb, 0)),
            pl.BlockSpec((Q_BLK, TOPK), lambda qb, kb: (qb, 0)),
        ],
        out_shape=[
            jax.ShapeDtypeStruct((Q, TOPK), jnp.float32),
            jax.ShapeDtypeStruct((Q, TOPK), jnp.int32),
        ],
        scratch_shapes=[
            pltpu.VMEM((Q_BLK, 128 * nkb), jnp.float32),
            pltpu.VMEM((Q_BLK, 128 * nkb), jnp.int32),
        ],
    )(qn, de)

    return vals, idxs


# speedup vs baseline: 1.7428x; 1.2088x over previous
import functools

import jax
import jax.numpy as jnp
from jax.experimental import pallas as pl
from jax.experimental.pallas import tpu as pltpu

TOPK = 8
BIGF = 1e9


def _linear_kernel(xm_ref, w_ref, b_ref, o_ref):
    o_ref[...] = jax.lax.dot_general(
        xm_ref[...], w_ref[...], (((1,), (1,)), ((), ())),
        preferred_element_type=jnp.float32) + b_ref[...]


def _topk_rounds(s, iif, n):
    vals, idxs = [], []
    for t in range(n):
        m = jnp.max(s, axis=1)
        am = jnp.min(jnp.where(s == m[:, None], iif, BIGF), axis=1)
        vals.append(m)
        idxs.append(am)
        if t < n - 1:
            s = jnp.where(iif == am[:, None], -jnp.inf, s)
    return jnp.stack(vals, axis=1), jnp.stack(idxs, axis=1)


def _main_kernel(qn_ref, d_ref, vals_ref, idxs_ref, cv_ref, ci_ref, *,
                 k_total, k_blk):
    kb = pl.program_id(1)
    nkb = pl.num_programs(1)

    q = qn_ref[...]
    dn = d_ref[...]
    s = jax.lax.dot_general(q, dn, (((1,), (1,)), ((), ())),
                            preferred_element_type=jnp.float32)
    iif = (jax.lax.broadcasted_iota(jnp.int32, s.shape, 1).astype(jnp.float32)
           + (kb * k_blk).astype(jnp.float32))
    s = jnp.where(iif < jnp.float32(k_total), s, -jnp.inf)

    bv, bi = _topk_rounds(s, iif, TOPK)
    qb_rows = s.shape[0]
    fill_v = jnp.full((qb_rows, 128 - TOPK), -jnp.inf, jnp.float32)
    off = pl.multiple_of(kb * 128, 128)
    cv_ref[:, pl.ds(off, 128)] = jnp.concatenate([bv, fill_v], axis=1)
    ci_ref[:, pl.ds(off, 128)] = jnp.concatenate([bi, fill_v], axis=1)

    @pl.when(kb == nkb - 1)
    def _emit():
        allv = cv_ref[...]
        alli = ci_ref[...]
        jj = jax.lax.broadcasted_iota(
            jnp.int32, allv.shape, 1).astype(jnp.float32)
        nv, ni = [], []
        for t in range(TOPK):
            m = jnp.max(allv, axis=1)
            p = jnp.min(jnp.where(allv == m[:, None], jj, BIGF), axis=1)
            sel = jj == p[:, None]
            nv.append(m)
            ni.append(jnp.min(jnp.where(sel, alli, BIGF), axis=1))
            if t < TOPK - 1:
                allv = jnp.where(sel, -jnp.inf, allv)
        vals_ref[...] = jnp.stack(nv, axis=1)
        idxs_ref[...] = jnp.stack(ni, axis=1).astype(jnp.int32)


@jax.jit
def kernel(x_target, W, b, document_embeddings):
    Q, S, D = x_target.shape
    K = document_embeddings.shape[0]

    Q_BLK = 256
    K_BLK = 4096
    kpad = ((K + K_BLK - 1) // K_BLK) * K_BLK
    nkb = kpad // K_BLK

    xm = jnp.mean(x_target, axis=1)

    q = pl.pallas_call(
        _linear_kernel,
        grid=(Q // 128,),
        in_specs=[
            pl.BlockSpec((128, D), lambda qb: (qb, 0)),
            pl.BlockSpec((D, D), lambda qb: (0, 0)),
            pl.BlockSpec((1, D), lambda qb: (0, 0)),
        ],
        out_specs=pl.BlockSpec((128, D), lambda qb: (qb, 0)),
        out_shape=jax.ShapeDtypeStruct((Q, D), jnp.float32),
    )(xm, W, b.reshape(1, D))

    qn = q / (jnp.linalg.norm(q, axis=-1, keepdims=True) + 1e-12)

    de = document_embeddings / (
        jnp.linalg.norm(document_embeddings, axis=-1, keepdims=True) + 1e-12)
    de = de / (jnp.linalg.norm(de, axis=-1, keepdims=True) + 1e-12)
    de = jnp.pad(de, ((0, kpad - K), (0, 0)))

    vals, idxs = pl.pallas_call(
        functools.partial(_main_kernel, k_total=K, k_blk=K_BLK),
        grid=(Q // Q_BLK, nkb),
        in_specs=[
            pl.BlockSpec((Q_BLK, D), lambda qb, kb: (qb, 0)),
            pl.BlockSpec((K_BLK, D), lambda qb, kb: (kb, 0)),
        ],
        out_specs=[
            pl.BlockSpec((Q_BLK, TOPK), lambda qb, kb: (qb, 0)),
            pl.BlockSpec((Q_BLK, TOPK), lambda qb, kb: (qb, 0)),
        ],
        out_shape=[
            jax.ShapeDtypeStruct((Q, TOPK), jnp.float32),
            jax.ShapeDtypeStruct((Q, TOPK), jnp.int32),
        ],
        scratch_shapes=[
            pltpu.VMEM((Q_BLK, 128 * nkb), jnp.float32),
            pltpu.VMEM((Q_BLK, 128 * nkb), jnp.float32),
        ],
    )(qn, de)

    return vals, idxs
